# baseline (speedup 1.0000x reference)
import jax
import jax.numpy as jnp
from jax import lax
from jax.experimental import pallas as pl
from jax.experimental.pallas import tpu as pltpu
from jax.experimental.pallas import tpu_sc as plsc

B = 1024
S = 200
D = 128
N = B * S
NC = 2
NS = 16
NW = NC * NS
TOK_PER_W = N // NW
T = 32
NBLK = TOK_PER_W // T
NSLOT = 5
NGRP = NBLK // NSLOT
PE_EXT = S + T


def _tpe_sc(xf, hf, df, pe200, htab_pad, dtab_pad):
    mesh = plsc.VectorSubcoreMesh(core_axis_name="c", subcore_axis_name="s")

    @pl.kernel(
        out_type=jax.ShapeDtypeStruct((N, D), jnp.float32),
        mesh=mesh,
        compiler_params=pltpu.CompilerParams(needs_layout_passes=False),
        scratch_types=[
            pltpu.VMEM((PE_EXT, D), jnp.float32),
            pltpu.VMEM((24, D), jnp.float32),
            pltpu.VMEM((8, D), jnp.float32),
            pltpu.VMEM((TOK_PER_W,), jnp.int32),
            pltpu.VMEM((TOK_PER_W,), jnp.int32),
            [pltpu.VMEM((T, D), jnp.float32)] * NSLOT,
            pltpu.VMEM_SHARED((NS, NSLOT, T, D), jnp.float32),
            [pltpu.SemaphoreType.DMA] * NSLOT,
            [pltpu.SemaphoreType.DMA] * NSLOT,
            [pltpu.SemaphoreType.DMA] * NSLOT,
            [pltpu.SemaphoreType.DMA] * NSLOT,
            pltpu.SemaphoreType.DMA,
        ],
    )
    def k(x_hbm, h_hbm, d_hbm, pe_hbm, htab_hbm, dtab_hbm, out_hbm,
          pe_v, htab, dtab, hv, dv, xv, sp, s_in, x_in, x_out, s_out, psem):
        sid = lax.axis_index("s")
        wid = sid * NC + lax.axis_index("c")
        tok_base = wid * TOK_PER_W

        def hbm_slice(blk):
            return pl.ds(tok_base + blk * T, T)

        def issue_hbm_in(slot, blk):
            pltpu.async_copy(x_hbm.at[hbm_slice(blk)], sp.at[sid, slot], s_in[slot])

        def wait_hbm_in(slot, blk):
            pltpu.make_async_copy(
                x_hbm.at[hbm_slice(blk)], sp.at[sid, slot], s_in[slot]).wait()

        def issue_cross_in(slot):
            pltpu.async_copy(sp.at[sid, slot], xv[slot], x_in[slot])

        def wait_cross_in(slot):
            pltpu.make_async_copy(sp.at[sid, slot], xv[slot], x_in[slot]).wait()

        def issue_cross_out(slot):
            pltpu.async_copy(xv[slot], sp.at[sid, slot], x_out[slot])

        def wait_cross_out(slot):
            pltpu.make_async_copy(xv[slot], sp.at[sid, slot], x_out[slot]).wait()

        def issue_hbm_out(slot, blk):
            pltpu.async_copy(sp.at[sid, slot], out_hbm.at[hbm_slice(blk)], s_out[slot])

        def wait_hbm_out(slot, blk):
            pltpu.make_async_copy(
                sp.at[sid, slot], out_hbm.at[hbm_slice(blk)], s_out[slot]).wait()

        pltpu.async_copy(pe_hbm, pe_v.at[pl.ds(0, S)], psem)
        pltpu.async_copy(pe_hbm.at[pl.ds(0, T)], pe_v.at[pl.ds(S, T)], psem)
        pltpu.async_copy(htab_hbm, htab, psem)
        pltpu.async_copy(dtab_hbm, dtab, psem)
        pltpu.async_copy(h_hbm.at[pl.ds(tok_base, TOK_PER_W)], hv, psem)
        pltpu.async_copy(d_hbm.at[pl.ds(tok_base, TOK_PER_W)], dv, psem)
        for blk0 in range(3):
            issue_hbm_in(blk0, blk0)
        pltpu.make_async_copy(pe_hbm, pe_v.at[pl.ds(0, S)], psem).wait()
        pltpu.make_async_copy(pe_hbm.at[pl.ds(0, T)], pe_v.at[pl.ds(S, T)], psem).wait()
        pltpu.make_async_copy(htab_hbm, htab, psem).wait()
        pltpu.make_async_copy(dtab_hbm, dtab, psem).wait()
        pltpu.make_async_copy(h_hbm.at[pl.ds(tok_base, TOK_PER_W)], hv, psem).wait()
        pltpu.make_async_copy(d_hbm.at[pl.ds(tok_base, TOK_PER_W)], dv, psem).wait()
        wait_hbm_in(0, 0)
        issue_cross_in(0)

        @pl.loop(0, NGRP, init_carry=0)
        def _(g, s0g):
            s0 = s0g
            for kk in range(NSLOT):
                blk = g * NSLOT + kk
                k3 = (kk + 3) % NSLOT
                k1 = (kk + 1) % NSLOT
                kp = (kk - 1) % NSLOT

                if kk < 2:
                    @pl.when(g >= 1)
                    def _():
                        wait_hbm_out(k3, blk - 2)

                    issue_hbm_in(k3, blk + 3)
                else:
                    @pl.when(g <= NGRP - 2)
                    def _():
                        wait_hbm_out(k3, blk - 2)
                        issue_hbm_in(k3, blk + 3)

                if kk < NSLOT - 1:
                    wait_hbm_in(k1, blk + 1)
                    issue_cross_in(k1)
                else:
                    @pl.when(g <= NGRP - 2)
                    def _():
                        wait_hbm_in(k1, blk + 1)
                        issue_cross_in(k1)

                wait_cross_in(kk)
                s0k = s0

                @pl.loop(0, T, step=2)
                def _(t):
                    for u in range(2):
                        s = s0k + t + u
                        for c in range(8):
                            sl = pl.ds(c * 16, 16)
                            xv[kk][t + u, sl] = xv[kk][t + u, sl] + pe_v[s, sl]

                for t0 in (0, 16):
                    off = blk * T + t0
                    h16 = hv[pl.ds(off, 16)]
                    d16 = dv[pl.ds(off, 16)]
                    tok16 = lax.iota(jnp.int32, 16) + t0
                    for c in range(32):
                        cvec = jnp.full((16,), c, jnp.int32)
                        hvals = plsc.load_gather(htab, [h16, cvec])
                        plsc.addupdate_scatter(xv[kk], [tok16, cvec], hvals)
                        dvals = plsc.load_gather(dtab, [d16, cvec])
                        plsc.addupdate_scatter(xv[kk], [tok16, cvec + 32], dvals)

                issue_cross_out(kk)

                if kk >= 1:
                    wait_cross_out(kp)
                    issue_hbm_out(kp, blk - 1)
                else:
                    @pl.when(g >= 1)
                    def _():
                        wait_cross_out(kp)
                        issue_hbm_out(kp, blk - 1)

                s0n = s0 + T
                s0 = jnp.where(s0n >= S, s0n - S, s0n)
            return s0

        wait_cross_out(NSLOT - 1)
        issue_hbm_out(NSLOT - 1, NBLK - 1)
        for kk in range(NSLOT):
            wait_hbm_out(kk, NBLK - NSLOT + kk)

    return k(xf, hf, df, pe200, htab_pad, dtab_pad)


def kernel(x, hours, days, pe, hour_encoding, day_encoding):
    pe200 = pe[0, :S]
    htab_pad = jnp.zeros((24, D), jnp.float32).at[:, :32].set(hour_encoding)
    dtab_pad = jnp.zeros((8, D), jnp.float32).at[:7, :32].set(day_encoding)
    out = _tpe_sc(
        x.reshape(N, D),
        hours.astype(jnp.int32).reshape(N),
        days.astype(jnp.int32).reshape(N),
        pe200,
        htab_pad,
        dtab_pad,
    )
    return out.reshape(B, S, D)


# speedup vs baseline: 1.0658x; 1.0658x over previous
import jax
import jax.numpy as jnp
from jax import lax
from jax.experimental import pallas as pl
from jax.experimental.pallas import tpu as pltpu
from jax.experimental.pallas import tpu_sc as plsc

B = 1024
S = 200
D = 128
N = B * S
NC = 2
NS = 16
NW = NC * NS
TOK_PER_W = N // NW
T = 32
NBLK = TOK_PER_W // T
NPOS = NBLK // 2
NSLOT = 5
NGRP = NPOS // NSLOT
PE_EXT = S + 2 * T


def _tpe_sc(xf, hf, df, pe200, htab_pad, dtab_pad):
    mesh = plsc.VectorSubcoreMesh(core_axis_name="c", subcore_axis_name="s")

    @pl.kernel(
        out_type=jax.ShapeDtypeStruct((N, D), jnp.float32),
        mesh=mesh,
        compiler_params=pltpu.CompilerParams(needs_layout_passes=False),
        scratch_types=[
            pltpu.VMEM((PE_EXT, D), jnp.float32),
            pltpu.VMEM((24, D), jnp.float32),
            pltpu.VMEM((8, D), jnp.float32),
            pltpu.VMEM((TOK_PER_W,), jnp.int32),
            pltpu.VMEM((TOK_PER_W,), jnp.int32),
            [pltpu.VMEM((T, D), jnp.float32)] * NSLOT,
            [pltpu.VMEM((T, D), jnp.float32)] * NSLOT,
            pltpu.VMEM_SHARED((NS, NSLOT, T, D), jnp.float32),
            [pltpu.SemaphoreType.DMA] * NSLOT,
            [pltpu.SemaphoreType.DMA] * NSLOT,
            [pltpu.SemaphoreType.DMA] * NSLOT,
            [pltpu.SemaphoreType.DMA] * NSLOT,
            [pltpu.SemaphoreType.DMA] * NSLOT,
            [pltpu.SemaphoreType.DMA] * NSLOT,
            pltpu.SemaphoreType.DMA,
        ],
    )
    def k(x_hbm, h_hbm, d_hbm, pe_hbm, htab_hbm, dtab_hbm, out_hbm,
          pe_v, htab, dtab, hv, dv, xv, yv, sp,
          s_in, x_in, x_out, s_out, b_in, b_out, psem):
        sid = lax.axis_index("s")
        wid = sid * NC + lax.axis_index("c")
        tok_base = wid * TOK_PER_W

        def hbm_slice(blk):
            return pl.ds(tok_base + blk * T, T)

        def issue_hbm_in(slot, pos):
            pltpu.async_copy(x_hbm.at[hbm_slice(2 * pos)], sp.at[sid, slot],
                             s_in[slot])

        def wait_hbm_in(slot, pos):
            pltpu.make_async_copy(x_hbm.at[hbm_slice(2 * pos)],
                                  sp.at[sid, slot], s_in[slot]).wait()

        def issue_cross_in(slot):
            pltpu.async_copy(sp.at[sid, slot], xv[slot], x_in[slot])

        def wait_cross_in(slot):
            pltpu.make_async_copy(sp.at[sid, slot], xv[slot], x_in[slot]).wait()

        def issue_cross_out(slot):
            pltpu.async_copy(xv[slot], sp.at[sid, slot], x_out[slot])

        def wait_cross_out(slot):
            pltpu.make_async_copy(xv[slot], sp.at[sid, slot], x_out[slot]).wait()

        def issue_hbm_out(slot, pos):
            pltpu.async_copy(sp.at[sid, slot], out_hbm.at[hbm_slice(2 * pos)],
                             s_out[slot])

        def wait_hbm_out(slot, pos):
            pltpu.make_async_copy(sp.at[sid, slot],
                                  out_hbm.at[hbm_slice(2 * pos)],
                                  s_out[slot]).wait()

        def issue_dir_in(slot, pos):
            pltpu.async_copy(x_hbm.at[hbm_slice(2 * pos + 1)], yv[slot],
                             b_in[slot])

        def wait_dir_in(slot, pos):
            pltpu.make_async_copy(x_hbm.at[hbm_slice(2 * pos + 1)], yv[slot],
                                  b_in[slot]).wait()

        def issue_dir_out(slot, pos):
            pltpu.async_copy(yv[slot], out_hbm.at[hbm_slice(2 * pos + 1)],
                             b_out[slot])

        def wait_dir_out(slot, pos):
            pltpu.make_async_copy(yv[slot], out_hbm.at[hbm_slice(2 * pos + 1)],
                                  b_out[slot]).wait()

        def compute(buf, blk, s0):
            @pl.loop(0, T, step=2)
            def _(t):
                for u in range(2):
                    s = s0 + t + u
                    for c in range(8):
                        sl = pl.ds(c * 16, 16)
                        buf[t + u, sl] = buf[t + u, sl] + pe_v[s, sl]

            for t0 in (0, 16):
                off = blk * T + t0
                h16 = hv[pl.ds(off, 16)]
                d16 = dv[pl.ds(off, 16)]
                tok16 = lax.iota(jnp.int32, 16) + t0
                for c in range(32):
                    cvec = jnp.full((16,), c, jnp.int32)
                    hvals = plsc.load_gather(htab, [h16, cvec])
                    plsc.addupdate_scatter(buf, [tok16, cvec], hvals)
                    dvals = plsc.load_gather(dtab, [d16, cvec])
                    plsc.addupdate_scatter(buf, [tok16, cvec + 32], dvals)

        pltpu.async_copy(pe_hbm, pe_v.at[pl.ds(0, S)], psem)
        pltpu.async_copy(pe_hbm.at[pl.ds(0, 2 * T)], pe_v.at[pl.ds(S, 2 * T)], psem)
        pltpu.async_copy(htab_hbm, htab, psem)
        pltpu.async_copy(dtab_hbm, dtab, psem)
        pltpu.async_copy(h_hbm.at[pl.ds(tok_base, TOK_PER_W)], hv, psem)
        pltpu.async_copy(d_hbm.at[pl.ds(tok_base, TOK_PER_W)], dv, psem)
        for p0 in range(3):
            issue_hbm_in(p0, p0)
            issue_dir_in(p0, p0)
        pltpu.make_async_copy(pe_hbm, pe_v.at[pl.ds(0, S)], psem).wait()
        pltpu.make_async_copy(pe_hbm.at[pl.ds(0, 2 * T)], pe_v.at[pl.ds(S, 2 * T)], psem).wait()
        pltpu.make_async_copy(htab_hbm, htab, psem).wait()
        pltpu.make_async_copy(dtab_hbm, dtab, psem).wait()
        pltpu.make_async_copy(h_hbm.at[pl.ds(tok_base, TOK_PER_W)], hv, psem).wait()
        pltpu.make_async_copy(d_hbm.at[pl.ds(tok_base, TOK_PER_W)], dv, psem).wait()
        wait_hbm_in(0, 0)
        issue_cross_in(0)

        @pl.loop(0, NGRP, init_carry=0)
        def _(g, s0g):
            s0 = s0g
            for kk in range(NSLOT):
                pos = g * NSLOT + kk
                k3 = (kk + 3) % NSLOT
                k1 = (kk + 1) % NSLOT
                kp = (kk - 1) % NSLOT

                if kk < 2:
                    @pl.when(g >= 1)
                    def _():
                        wait_hbm_out(k3, pos - 2)
                        wait_dir_out(k3, pos - 2)

                    issue_hbm_in(k3, pos + 3)
                    issue_dir_in(k3, pos + 3)
                else:
                    @pl.when(g <= NGRP - 2)
                    def _():
                        wait_hbm_out(k3, pos - 2)
                        wait_dir_out(k3, pos - 2)
                        issue_hbm_in(k3, pos + 3)
                        issue_dir_in(k3, pos + 3)

                if kk < NSLOT - 1:
                    wait_hbm_in(k1, pos + 1)
                    issue_cross_in(k1)
                else:
                    @pl.when(g <= NGRP - 2)
                    def _():
                        wait_hbm_in(k1, pos + 1)
                        issue_cross_in(k1)

                wait_cross_in(kk)
                compute(xv[kk], 2 * pos, s0)
                issue_cross_out(kk)

                wait_dir_in(kk, pos)
                compute(yv[kk], 2 * pos + 1, s0 + T)
                issue_dir_out(kk, pos)

                if kk >= 1:
                    wait_cross_out(kp)
                    issue_hbm_out(kp, pos - 1)
                else:
                    @pl.when(g >= 1)
                    def _():
                        wait_cross_out(kp)
                        issue_hbm_out(kp, pos - 1)

                s0n = s0 + 2 * T
                s0 = jnp.where(s0n >= S, s0n - S, s0n)
            return s0

        wait_cross_out(NSLOT - 1)
        issue_hbm_out(NSLOT - 1, NPOS - 1)
        for kk in range(NSLOT):
            wait_hbm_out(kk, NPOS - NSLOT + kk)
            wait_dir_out(kk, NPOS - NSLOT + kk)

    return k(xf, hf, df, pe200, htab_pad, dtab_pad)


def kernel(x, hours, days, pe, hour_encoding, day_encoding):
    pe200 = pe[0, :S]
    htab_pad = jnp.zeros((24, D), jnp.float32).at[:, :32].set(hour_encoding)
    dtab_pad = jnp.zeros((8, D), jnp.float32).at[:7, :32].set(day_encoding)
    out = _tpe_sc(
        x.reshape(N, D),
        hours.astype(jnp.int32).reshape(N),
        days.astype(jnp.int32).reshape(N),
        pe200,
        htab_pad,
        dtab_pad,
    )
    return out.reshape(B, S, D)
